# Initial kernel scaffold; baseline (speedup 1.0000x reference)
import jax
import jax.numpy as jnp
from jax.experimental import pallas as pl


def kernel(x, pos_table):
    raise NotImplementedError("write your pallas kernel here")


# speedup vs baseline: 1.4905x; 1.4905x over previous
import jax
import jax.numpy as jnp
from jax.experimental import pallas as pl

S_BLK = 512


def _body(x_ref, p_ref, o_ref):
    o_ref[...] = x_ref[...] + p_ref[...]


def kernel(x, pos_table):
    B, S, D = x.shape
    n_s = S // S_BLK
    return pl.pallas_call(
        _body,
        grid=(n_s, B),
        in_specs=[
            pl.BlockSpec((1, S_BLK, D), lambda i, j: (j, i, 0)),
            pl.BlockSpec((1, S_BLK, D), lambda i, j: (0, i, 0)),
        ],
        out_specs=pl.BlockSpec((1, S_BLK, D), lambda i, j: (j, i, 0)),
        out_shape=jax.ShapeDtypeStruct(x.shape, x.dtype),
    )(x, pos_table[None, :S, :])


# speedup vs baseline: 1.6688x; 1.1196x over previous
import jax
import jax.numpy as jnp
from jax.experimental import pallas as pl

S_BLK = 1024


def _body(x_ref, p_ref, o_ref):
    o_ref[...] = x_ref[...] + p_ref[...]


def kernel(x, pos_table):
    B, S, D = x.shape
    n_s = S // S_BLK
    return pl.pallas_call(
        _body,
        grid=(n_s, B),
        in_specs=[
            pl.BlockSpec((1, S_BLK, D), lambda i, j: (j, i, 0)),
            pl.BlockSpec((1, S_BLK, D), lambda i, j: (0, i, 0)),
        ],
        out_specs=pl.BlockSpec((1, S_BLK, D), lambda i, j: (j, i, 0)),
        out_shape=jax.ShapeDtypeStruct(x.shape, x.dtype),
    )(x, pos_table[None, :S, :])


# speedup vs baseline: 1.7383x; 1.0416x over previous
import jax
import jax.numpy as jnp
from jax.experimental import pallas as pl

S_BLK = 2048


def _body(x_ref, p_ref, o_ref):
    o_ref[...] = x_ref[...] + p_ref[...]


def kernel(x, pos_table):
    B, S, D = x.shape
    n_s = S // S_BLK
    return pl.pallas_call(
        _body,
        grid=(n_s, B),
        in_specs=[
            pl.BlockSpec((1, S_BLK, D), lambda i, j: (j, i, 0)),
            pl.BlockSpec((1, S_BLK, D), lambda i, j: (0, i, 0)),
        ],
        out_specs=pl.BlockSpec((1, S_BLK, D), lambda i, j: (j, i, 0)),
        out_shape=jax.ShapeDtypeStruct(x.shape, x.dtype),
    )(x, pos_table[None, :S, :])
